# Initial kernel scaffold; baseline (speedup 1.0000x reference)
import jax
import jax.numpy as jnp
from jax.experimental import pallas as pl


def kernel(pred_logits, pred_boxes, target_sizes):
    raise NotImplementedError("write your pallas kernel here")


# speedup vs baseline: 6.1329x; 6.1329x over previous
import jax
import jax.numpy as jnp
from jax.experimental import pallas as pl
from jax.experimental.pallas import tpu as pltpu

_B, _Q, _C = 8, 20000, 92
_CM1 = _C - 1
_K = 100
_QB = 2000
_NQ = _Q // _QB
_PAD = 104
_BIG = 2**30


def _stats_body(x_ref, r_ref):
    x = x_ref[0]
    m = jnp.max(x, axis=-1)
    e = jnp.exp(x - m[:, None])
    s = jnp.sum(e, axis=-1)
    m91 = jnp.max(x[:, :_CM1], axis=-1)
    r_ref[0, 0, :] = jnp.exp(m91 - m) / s


def _rowtopk_body(r_ref, cand_ref):
    x = r_ref[...]
    iq = jax.lax.broadcasted_iota(jnp.int32, (_B, _Q), 1)
    lane = jax.lax.broadcasted_iota(jnp.int32, (_B, 128), 1)

    def body(t, carry):
        x, cand = carry
        vmax = jnp.max(x, axis=1, keepdims=True)
        qidx = jnp.min(jnp.where(x >= vmax, iq, _BIG), axis=1,
                       keepdims=True)
        cand = jnp.where(lane == t, qidx, cand)
        x = jnp.where(iq == qidx, -jnp.inf, x)
        return x, cand

    _, cand = jax.lax.fori_loop(
        0, _K, body, (x, jnp.zeros((_B, 128), jnp.int32)))
    cand_ref[...] = cand


def _final_body(cand_ref, ts_ref, x_ref, bx_ref, sc_ref, lb_ref, bo_ref,
                probs_ref, boxes_ref):
    b = pl.program_id(0)
    probs_ref[...] = jnp.full((_PAD, _C), -1.0, jnp.float32)
    lanec = jax.lax.broadcasted_iota(jnp.int32, (1, _C), 1)
    sub = jax.lax.broadcasted_iota(jnp.int32, (_PAD, 1), 0)

    def gbody(j, qcol):
        q = cand_ref[b, j]
        row = x_ref[0, pl.ds(q, 1), :]
        m = jnp.max(row)
        e = jnp.exp(row - m)
        s = jnp.sum(e)
        probs_ref[pl.ds(j, 1), :] = jnp.where(lanec < _CM1, e / s, -1.0)
        return jnp.where(sub == j, q, qcol)

    qcol = jax.lax.fori_loop(0, _K, gbody, jnp.zeros((_PAD, 1), jnp.int32))

    x = probs_ref[...]
    c2 = jax.lax.broadcasted_iota(jnp.int32, (_PAD, _C), 1)
    fl = qcol * _CM1 + c2
    lane128 = jax.lax.broadcasted_iota(jnp.int32, (1, 128), 1)

    def tbody(t, carry):
        x, sv, lv = carry
        vmax = jnp.max(x)
        fsel = jnp.min(jnp.where(x >= vmax, fl, _BIG))
        sv = jnp.where(lane128 == t, vmax, sv)
        lv = jnp.where(lane128 == t, fsel % _CM1, lv)
        bq = fsel // _CM1
        boxes_ref[pl.ds(t, 1), :] = bx_ref[0, pl.ds(bq, 1), :]
        x = jnp.where(fl == fsel, -1.0, x)
        return x, sv, lv

    _, sv, lv = jax.lax.fori_loop(
        0, _K, tbody,
        (x, jnp.zeros((1, 128), jnp.float32), jnp.zeros((1, 128), jnp.int32)))

    raw = boxes_ref[...]
    cx, cy, w, h = raw[:, 0:1], raw[:, 1:2], raw[:, 2:3], raw[:, 3:4]
    xyxy = jnp.concatenate(
        [cx - 0.5 * w, cy - 0.5 * h, cx + 0.5 * w, cy + 0.5 * h], axis=1)
    hf = ts_ref[b, 0].astype(jnp.float32)
    wf = ts_ref[b, 1].astype(jnp.float32)
    lane4 = jax.lax.broadcasted_iota(jnp.int32, (_PAD, 4), 1)
    scale = jnp.where(lane4 % 2 == 0, wf, hf)
    sc_ref[0, 0, :] = sv[0]
    lb_ref[0, 0, :] = lv[0]
    bo_ref[0] = xyxy * scale


def kernel(pred_logits, pred_boxes, target_sizes):
    r3 = pl.pallas_call(
        _stats_body,
        grid=(_B, _NQ),
        in_specs=[pl.BlockSpec((1, _QB, _C), lambda b, j: (b, j, 0))],
        out_specs=pl.BlockSpec((1, 1, _QB), lambda b, j: (b * _NQ + j, 0, 0)),
        out_shape=jax.ShapeDtypeStruct((_B * _NQ, 1, _QB), jnp.float32),
    )(pred_logits)
    rowkey = r3.reshape(_B, _Q)

    cand = pl.pallas_call(
        _rowtopk_body,
        out_shape=jax.ShapeDtypeStruct((_B, 128), jnp.int32),
    )(rowkey)

    scores, labels, boxes = pl.pallas_call(
        _final_body,
        grid=(_B,),
        in_specs=[
            pl.BlockSpec(memory_space=pltpu.SMEM),
            pl.BlockSpec(memory_space=pltpu.SMEM),
            pl.BlockSpec((1, _Q, _C), lambda b: (b, 0, 0)),
            pl.BlockSpec((1, _Q, 4), lambda b: (b, 0, 0)),
        ],
        out_specs=[
            pl.BlockSpec((1, 1, 128), lambda b: (b, 0, 0)),
            pl.BlockSpec((1, 1, 128), lambda b: (b, 0, 0)),
            pl.BlockSpec((1, _PAD, 4), lambda b: (b, 0, 0)),
        ],
        out_shape=[
            jax.ShapeDtypeStruct((_B, 1, 128), jnp.float32),
            jax.ShapeDtypeStruct((_B, 1, 128), jnp.int32),
            jax.ShapeDtypeStruct((_B, _PAD, 4), jnp.float32),
        ],
        scratch_shapes=[
            pltpu.VMEM((_PAD, _C), jnp.float32),
            pltpu.VMEM((_PAD, 4), jnp.float32),
        ],
    )(cand, target_sizes, pred_logits, pred_boxes)

    return scores[:, 0, :_K], labels[:, 0, :_K], boxes[:, :_K, :]


# speedup vs baseline: 8.8233x; 1.4387x over previous
import jax
import jax.numpy as jnp
from jax import lax
from jax.experimental import pallas as pl
from jax.experimental.pallas import tpu as pltpu
from jax.experimental.pallas import tpu_sc as plsc

_B, _Q, _C = 8, 20000, 92
_CM1 = _C - 1
_K = 100
_QB = 2000
_NQ = _Q // _QB
_RCAP = 128
_ECAP = 512
_BIG = 2**30
_MARGIN = 1.0 - 1e-5
_L = 16
_NSL = 6
_CP = 96
_BP = 16


def _stats_body(x_ref, r_ref, p_ref):
    x = x_ref[0]
    m = jnp.max(x, axis=-1)
    e = jnp.exp(x - m[:, None])
    s = jnp.sum(e, axis=-1)
    m91 = jnp.max(x[:, :_CM1], axis=-1)
    r_ref[0, 0, :] = jnp.exp(m91 - m) / s
    p_ref[0] = jnp.concatenate(
        [x, jnp.zeros((_QB, _CP - _C), jnp.float32)], axis=1)


def _thresh_body(r_ref, t_ref):
    x = r_ref[...]

    def body(t, carry):
        lo, hi = carry
        mid = (lo + hi) // 2
        tf = lax.bitcast_convert_type(mid, jnp.float32)
        cnt = jnp.sum((x >= tf).astype(jnp.int32), axis=1, keepdims=True)
        pred = cnt >= _K
        return jnp.where(pred, mid, lo), jnp.where(pred, hi, mid)

    lo, _ = lax.fori_loop(
        0, 31, body,
        (jnp.zeros((_B, 1), jnp.int32),
         jnp.full((_B, 1), 0x3F800001, jnp.int32)))
    v100 = lax.bitcast_convert_type(lo, jnp.float32)
    t_ref[...] = jnp.broadcast_to(v100 * _MARGIN, (_B, 128))


def _sc_body(rk_hbm, te_hbm, lg_hbm, bx_hbm,
             vals_hbm, fidx_hbm, cbox_hbm,
             rk_v, candq_v, gidx_v, rows_v, brows_v, te_v, vals_v, fidx_v,
             sem):
    b = lax.axis_index("s") * 2 + lax.axis_index("c")

    @pl.when(b < _B)
    def _():
        pltpu.sync_copy(rk_hbm.at[b], rk_v)
        pltpu.sync_copy(te_hbm.at[b], te_v)
        lane = lax.broadcasted_iota(jnp.int32, (_L,), 0)
        tspl = te_v[...]

        neg1 = jnp.full((_L,), -1, jnp.int32)
        for k in range(_RCAP // _L):
            candq_v[pl.ds(k * _L, _L)] = neg1

        def scan_body(i, base):
            xv = rk_v[pl.ds(i * _L, _L)]
            m = xv >= tspl
            pos = base + plsc.cumsum(m.astype(jnp.int32)) - 1
            ok = m & (pos < _RCAP)
            plsc.store_scatter(candq_v, [jnp.where(ok, pos, 0)],
                               lane + i * _L, mask=ok)
            return base + plsc.all_reduce_population_count(m)

        nrow = lax.fori_loop(0, _Q // _L, scan_body,
                             jnp.zeros((_L,), jnp.int32))

        for k in range(_RCAP // _L):
            cq = candq_v[pl.ds(k * _L, _L)]
            gidx_v[pl.ds(k * _L, _L)] = jnp.where(cq >= 0, cq + b * _Q, 0)

        pltpu.async_copy(lg_hbm.at[gidx_v], rows_v, sem).wait()
        pltpu.async_copy(bx_hbm.at[gidx_v], brows_v, sem).wait()

        for k in range(_ECAP // _L):
            vals_v[pl.ds(k * _L, _L)] = jnp.full((_L,), -1.0, jnp.float32)
            fidx_v[pl.ds(k * _L, _L)] = jnp.full((_L,), _BIG, jnp.int32)

        def row_body(j, ebase):
            jf = jnp.full((_L,), j, jnp.int32)
            xs = []
            for k in range(_NSL):
                cidx = lane + k * _L
                xv = plsc.load_gather(rows_v, [jf, cidx])
                xs.append((xv, cidx))
            m = jnp.full((_L,), -3.0e38, jnp.float32)
            for xv, cidx in xs:
                m = jnp.maximum(m, jnp.where(cidx < _C, xv, -3.0e38))
            ms = jnp.max(m)
            sv = jnp.zeros((_L,), jnp.float32)
            es = []
            for xv, cidx in xs:
                e = jnp.where(cidx < _C, jnp.exp(xv - ms), 0.0)
                es.append((e, cidx))
                sv = sv + e
            inv = jnp.full((_L,), 1.0, jnp.float32) / jnp.sum(sv)
            jvalid = jf < nrow
            for e, cidx in es:
                p = e * inv
                okm = (p >= tspl) & (cidx < _CM1) & jvalid
                pos = ebase + plsc.cumsum(okm.astype(jnp.int32)) - 1
                ok2 = okm & (pos < _ECAP)
                posc = jnp.where(ok2, pos, 0)
                plsc.store_scatter(vals_v, [posc], p, mask=ok2)
                plsc.store_scatter(fidx_v, [posc], jf * _CM1 + cidx, mask=ok2)
                ebase = ebase + plsc.all_reduce_population_count(okm)
            return ebase

        lax.fori_loop(0, _RCAP, row_body, jnp.zeros((_L,), jnp.int32))

        pltpu.sync_copy(vals_v, vals_hbm.at[b])
        pltpu.sync_copy(fidx_v, fidx_hbm.at[b])
        pltpu.sync_copy(brows_v, cbox_hbm.at[b])


def _final_body(vals_ref, fidx_ref, cbox_ref, ts_ref, sc_ref, lb_ref, bo_ref):
    x = vals_ref[...]
    fx = fidx_ref[...]
    cb = cbox_ref[...]
    lane128 = lax.broadcasted_iota(jnp.int32, (_B, 128), 1)
    jio = lax.broadcasted_iota(jnp.int32, (_B, _RCAP, 4), 1)

    def body(t, carry):
        x, sv, lv, bacc = carry
        vmax = jnp.max(x, axis=1, keepdims=True)
        fsel = jnp.min(jnp.where(x >= vmax, fx, _BIG), axis=1,
                       keepdims=True)
        sv = jnp.where(lane128 == t, vmax, sv)
        lv = jnp.where(lane128 == t, fsel % _CM1, lv)
        jsel = (fsel // _CM1)[:, :, None]
        bt = jnp.sum(jnp.where(jio == jsel, cb, 0.0), axis=1,
                     keepdims=True)
        bacc = jnp.where(jio == t, bt, bacc)
        x = jnp.where(fx == fsel, -1.0, x)
        return x, sv, lv, bacc

    _, sv, lv, bacc = lax.fori_loop(
        0, _K, body,
        (x, jnp.zeros((_B, 128), jnp.float32),
         jnp.zeros((_B, 128), jnp.int32),
         jnp.zeros((_B, _RCAP, 4), jnp.float32)))

    cx, cy = bacc[:, :, 0:1], bacc[:, :, 1:2]
    w, h = bacc[:, :, 2:3], bacc[:, :, 3:4]
    xyxy = jnp.concatenate(
        [cx - 0.5 * w, cy - 0.5 * h, cx + 0.5 * w, cy + 0.5 * h], axis=2)
    tsv = ts_ref[...]
    iw, ih = tsv[:, 1:2], tsv[:, 0:1]
    scale = jnp.concatenate([iw, ih, iw, ih], axis=1)[:, None, :]
    sc_ref[...] = sv
    lb_ref[...] = lv
    bo_ref[...] = xyxy * scale


def kernel(pred_logits, pred_boxes, target_sizes):
    r3, lg96 = pl.pallas_call(
        _stats_body,
        grid=(_B, _NQ),
        in_specs=[pl.BlockSpec((1, _QB, _C), lambda b, j: (b, j, 0))],
        out_specs=[
            pl.BlockSpec((1, 1, _QB), lambda b, j: (b * _NQ + j, 0, 0)),
            pl.BlockSpec((1, _QB, _CP), lambda b, j: (b, j, 0)),
        ],
        out_shape=[
            jax.ShapeDtypeStruct((_B * _NQ, 1, _QB), jnp.float32),
            jax.ShapeDtypeStruct((_B, _Q, _CP), jnp.float32),
        ],
    )(pred_logits)
    rowkey = r3.reshape(_B, _Q)

    teff = pl.pallas_call(
        _thresh_body,
        out_shape=jax.ShapeDtypeStruct((_B, 128), jnp.float32),
    )(rowkey)
    teff16 = jnp.broadcast_to(teff[:, :1], (_B, _L))

    sc_fn = pl.kernel(
        _sc_body,
        mesh=plsc.VectorSubcoreMesh(core_axis_name="c", subcore_axis_name="s"),
        compiler_params=pltpu.CompilerParams(
            needs_layout_passes=False, use_tc_tiling_on_sc=False),
        out_type=[
            jax.ShapeDtypeStruct((_B, _ECAP), jnp.float32),
            jax.ShapeDtypeStruct((_B, _ECAP), jnp.int32),
            jax.ShapeDtypeStruct((_B, _RCAP, _BP), jnp.float32),
        ],
        scratch_types=[
            pltpu.VMEM((_Q,), jnp.float32),
            pltpu.VMEM((_RCAP,), jnp.int32),
            pltpu.VMEM((_RCAP,), jnp.int32),
            pltpu.VMEM((_RCAP, _CP), jnp.float32),
            pltpu.VMEM((_RCAP, _BP), jnp.float32),
            pltpu.VMEM((_L,), jnp.float32),
            pltpu.VMEM((_ECAP,), jnp.float32),
            pltpu.VMEM((_ECAP,), jnp.int32),
            pltpu.SemaphoreType.DMA,
        ],
    )
    bx16 = jnp.pad(pred_boxes.reshape(_B * _Q, 4), ((0, 0), (0, _BP - 4)))
    vals, fidx, cbox = sc_fn(
        rowkey, teff16, lg96.reshape(_B * _Q, _CP), bx16)

    scores, labels, boxes = pl.pallas_call(
        _final_body,
        out_shape=[
            jax.ShapeDtypeStruct((_B, 128), jnp.float32),
            jax.ShapeDtypeStruct((_B, 128), jnp.int32),
            jax.ShapeDtypeStruct((_B, _RCAP, 4), jnp.float32),
        ],
    )(vals, fidx, cbox[:, :, :4], target_sizes.astype(jnp.float32))

    return scores[:, :_K], labels[:, :_K], boxes[:, :_K, :]
